# baseline (speedup 1.0000x reference)
import functools

import jax
import jax.numpy as jnp
from jax import lax
from jax.experimental import pallas as pl
from jax.experimental.pallas import tpu as pltpu
from jax.experimental.pallas import tpu_sc as plsc

BATCH = 16384
NF = 32
CHUNK = 128


def kernel(user_idx, item_idx, user_embedding, item_embedding, user_bias, item_bias):
    info = plsc.get_sparse_core_info()
    NC, NS, L = info.num_cores, info.num_subcores, info.num_lanes
    NW = NC * NS
    b_per_w = BATCH // NW
    n_chunks = b_per_w // CHUNK
    n_groups = b_per_w // L

    mesh = plsc.VectorSubcoreMesh(core_axis_name="c", subcore_axis_name="s")

    @functools.partial(
        pl.kernel,
        mesh=mesh,
        out_type=jax.ShapeDtypeStruct((BATCH,), jnp.float32),
        compiler_params=pltpu.CompilerParams(
            needs_layout_passes=False, use_tc_tiling_on_sc=False),
        scratch_types=[
            pltpu.VMEM((n_chunks, CHUNK), jnp.int32),
            pltpu.VMEM((n_chunks, CHUNK), jnp.int32),
            pltpu.VMEM((b_per_w, NF), jnp.float32),
            pltpu.VMEM((b_per_w, NF), jnp.float32),
            pltpu.VMEM((b_per_w,), jnp.float32),
            pltpu.VMEM((b_per_w,), jnp.float32),
            pltpu.VMEM((b_per_w,), jnp.float32),
            pltpu.SemaphoreType.DMA,
        ],
    )
    def sc_kernel(uidx_hbm, iidx_hbm, uemb_hbm, iemb_hbm, ub_hbm, ib_hbm,
                  out_hbm, uidx_v, iidx_v, urows_v, irows_v, ub_v, ib_v,
                  out_v, sem):
        wid = lax.axis_index("s") * NC + lax.axis_index("c")
        base = wid * b_per_w

        for c in range(n_chunks):
            pltpu.sync_copy(uidx_hbm.at[pl.ds(base + c * CHUNK, CHUNK)],
                            uidx_v.at[c])
            pltpu.sync_copy(iidx_hbm.at[pl.ds(base + c * CHUNK, CHUNK)],
                            iidx_v.at[c])

        copies = []
        for c in range(n_chunks):
            dst = pl.ds(c * CHUNK, CHUNK)
            copies.append(pltpu.async_copy(
                uemb_hbm.at[uidx_v.at[c]], urows_v.at[dst, :], sem))
            copies.append(pltpu.async_copy(
                iemb_hbm.at[iidx_v.at[c]], irows_v.at[dst, :], sem))
            copies.append(pltpu.async_copy(
                ub_hbm.at[uidx_v.at[c]], ub_v.at[dst], sem))
            copies.append(pltpu.async_copy(
                ib_hbm.at[iidx_v.at[c]], ib_v.at[dst], sem))
        for cp in copies:
            cp.wait()

        lane = lax.iota(jnp.int32, L)

        def group_body(g, carry):
            row = g * L + lane
            acc = (plsc.load_gather(ub_v, [row])
                   + plsc.load_gather(ib_v, [row]))
            for kf in range(NF):
                col = jnp.full((L,), kf, jnp.int32)
                acc = acc + (plsc.load_gather(urows_v, [row, col])
                             * plsc.load_gather(irows_v, [row, col]))
            out_v[pl.ds(g * L, L)] = acc
            return carry

        lax.fori_loop(0, n_groups, group_body, 0)

        pltpu.sync_copy(out_v, out_hbm.at[pl.ds(base, b_per_w)])

    return sc_kernel(user_idx, item_idx, user_embedding, item_embedding,
                     user_bias.reshape(-1), item_bias.reshape(-1))


# speedup vs baseline: 1.9116x; 1.9116x over previous
import functools

import jax
import jax.numpy as jnp
from jax import lax
from jax.experimental import pallas as pl
from jax.experimental.pallas import tpu as pltpu
from jax.experimental.pallas import tpu_sc as plsc

BATCH = 16384
NF = 32
CHUNK = 128
WB = 16
SEL_CAP = 1040
IDX_CHUNK = 2048
RING = 16

_info = plsc.get_sparse_core_info()
_NC, _NS, _L = _info.num_cores, _info.num_subcores, _info.num_lanes
_NW = _NC * _NS


def _sweep_kernel(n_users, n_items):
    nt_u = -(-n_users // CHUNK)
    nt_i = -(-n_items // CHUNK)
    wpw_u = -(-nt_u // _NW)
    wpw_i = -(-nt_i // _NW)
    nb_u = -(-wpw_u // WB)
    nb_i = -(-wpw_i // WB)
    mesh = plsc.VectorSubcoreMesh(core_axis_name="c", subcore_axis_name="s")

    @functools.partial(
        pl.kernel,
        mesh=mesh,
        out_type=(jax.ShapeDtypeStruct((BATCH * NF,), jnp.float32),
                  jax.ShapeDtypeStruct((BATCH * NF,), jnp.float32)),
        compiler_params=pltpu.CompilerParams(
            needs_layout_passes=False, use_tc_tiling_on_sc=True),
        scratch_types=[
            pltpu.VMEM((IDX_CHUNK,), jnp.int32),
            pltpu.VMEM((SEL_CAP,), jnp.int32),
            pltpu.VMEM((SEL_CAP,), jnp.int32),
            pltpu.VMEM((NF, WB * CHUNK), jnp.float32),
            pltpu.VMEM((RING * NF,), jnp.float32),
            pltpu.SMEM((8,), jnp.int32),
            pltpu.SemaphoreType.DMA,
            pltpu.SemaphoreType.DMA,
        ],
    )
    def k1(uidx_hbm, iidx_hbm, uT3_hbm, iT3_hbm, ru_hbm, ri_hbm,
           idx_v, seln_v, selb_v, stage_v, rb_v, cnt_s, sem, sem2):
        wid = lax.axis_index("s") * _NC + lax.axis_index("c")
        lane = lax.iota(jnp.int32, _L)

        def sweep(idx_hbm, tbl3, nt, wpw, nb, rout):
            lo = wid * wpw
            hi = jnp.minimum(nt, lo + wpw)

            def sel_chunk(ci, cnt):
                pltpu.sync_copy(idx_hbm.at[pl.ds(ci * IDX_CHUNK, IDX_CHUNK)],
                                idx_v)

                def sel_vec(j, cnt):
                    nv = idx_v[pl.ds(j * _L, _L)]
                    wv = nv >> 7
                    m = (wv >= lo) & (wv < hi)
                    plsc.store_compressed(seln_v.at[pl.ds(cnt, _L)], nv, mask=m)
                    bv = ci * IDX_CHUNK + j * _L + lane
                    plsc.store_compressed(selb_v.at[pl.ds(cnt, _L)], bv, mask=m)
                    return cnt + plsc.all_reduce_population_count(m)[0]

                return lax.fori_loop(0, IDX_CHUNK // _L, sel_vec, cnt)

            cnt = lax.fori_loop(0, BATCH // IDX_CHUNK, sel_chunk, 0)

            def batch_body(bi, carry):
                win0 = lo + bi * WB
                win0c = jnp.minimum(win0, nt - WB)
                col0 = pl.multiple_of(win0c * CHUNK, CHUNK)
                cps = [pltpu.async_copy(
                    tbl3.at[ft, :, pl.ds(col0, WB * CHUNK)],
                    stage_v.at[pl.ds(ft * 8, 8), :], sem2)
                    for ft in range(NF // 8)]
                for cp in cps:
                    cp.wait()
                wend = jnp.minimum(win0 + WB, hi)

                def scan_vec(v, carry):
                    nv = seln_v[pl.ds(v * _L, _L)]
                    bv = selb_v[pl.ds(v * _L, _L)]
                    wv = nv >> 7
                    m = ((v * _L + lane) < cnt) & (wv >= win0) & (wv < wend)
                    nhits = plsc.all_reduce_population_count(m)[0]
                    mi = m.astype(jnp.int32)

                    @pl.when(nhits > 0)
                    def _():
                        for k in range(_L):
                            @pl.when(mi[k] != 0)
                            def _():
                                n_s = nv[k]
                                b_s = bv[k]
                                col = jnp.full((_L,), n_s - win0c * CHUNK,
                                               jnp.int32)
                                r0 = plsc.load_gather(stage_v, [lane, col])
                                r1 = plsc.load_gather(stage_v,
                                                      [lane + _L, col])
                                h = cnt_s[0]

                                @pl.when(h >= RING)
                                def _():
                                    pltpu.make_async_copy(
                                        idx_hbm.at[pl.ds(0, NF)],
                                        rb_v.at[pl.ds(0, NF)], sem).wait()

                                off = (h & (RING - 1)) * NF
                                rb_v[pl.ds(off, _L)] = r0
                                rb_v[pl.ds(off + _L, _L)] = r1
                                pltpu.async_copy(
                                    rb_v.at[pl.ds(off, NF)],
                                    rout.at[pl.ds(b_s * NF, NF)], sem)
                                cnt_s[0] = h + 1
                    return carry

                return lax.fori_loop(0, SEL_CAP // _L, scan_vec, carry)

            lax.fori_loop(0, nb, batch_body, 0)

        cnt_s[0] = 0
        sweep(iidx_hbm, iT3_hbm, nt_i, wpw_i, nb_i, ri_hbm)
        sweep(uidx_hbm, uT3_hbm, nt_u, wpw_u, nb_u, ru_hbm)

        total = cnt_s[0]

        def drain_body(i, carry):
            @pl.when(i < jnp.minimum(total, RING))
            def _():
                pltpu.make_async_copy(uidx_hbm.at[pl.ds(0, NF)],
                                      rb_v.at[pl.ds(0, NF)], sem).wait()
            return carry

        lax.fori_loop(0, RING, drain_body, 0)

    return k1


def _combine_kernel():
    b_per_w = BATCH // _NW
    n_chunks = b_per_w // CHUNK
    n_groups = b_per_w // _L
    mesh = plsc.VectorSubcoreMesh(core_axis_name="c", subcore_axis_name="s")

    @functools.partial(
        pl.kernel,
        mesh=mesh,
        out_type=jax.ShapeDtypeStruct((BATCH,), jnp.float32),
        compiler_params=pltpu.CompilerParams(needs_layout_passes=False),
        scratch_types=[
            pltpu.VMEM((n_chunks, CHUNK), jnp.int32),
            pltpu.VMEM((n_chunks, CHUNK), jnp.int32),
            pltpu.VMEM((b_per_w * NF,), jnp.float32),
            pltpu.VMEM((b_per_w * NF,), jnp.float32),
            pltpu.VMEM((b_per_w,), jnp.float32),
            pltpu.VMEM((b_per_w,), jnp.float32),
            pltpu.VMEM((b_per_w,), jnp.float32),
            pltpu.SemaphoreType.DMA,
        ],
    )
    def k2(uidx_hbm, iidx_hbm, ru_hbm, ri_hbm, ub_hbm, ib_hbm, out_hbm,
           uidx_v, iidx_v, ur_v, ir_v, ub_v, ib_v, out_v, sem):
        wid = lax.axis_index("s") * _NC + lax.axis_index("c")
        base = wid * b_per_w

        for c in range(n_chunks):
            pltpu.sync_copy(uidx_hbm.at[pl.ds(base + c * CHUNK, CHUNK)],
                            uidx_v.at[c])
            pltpu.sync_copy(iidx_hbm.at[pl.ds(base + c * CHUNK, CHUNK)],
                            iidx_v.at[c])

        copies = [
            pltpu.async_copy(ru_hbm.at[pl.ds(base * NF, b_per_w * NF)],
                             ur_v, sem),
            pltpu.async_copy(ri_hbm.at[pl.ds(base * NF, b_per_w * NF)],
                             ir_v, sem),
        ]
        for c in range(n_chunks):
            dst = pl.ds(c * CHUNK, CHUNK)
            copies.append(pltpu.async_copy(
                ub_hbm.at[uidx_v.at[c]], ub_v.at[dst], sem))
            copies.append(pltpu.async_copy(
                ib_hbm.at[iidx_v.at[c]], ib_v.at[dst], sem))
        for cp in copies:
            cp.wait()

        lane = lax.iota(jnp.int32, _L)

        def dot_body(g, carry):
            row = g * _L + lane
            acc = plsc.load_gather(ub_v, [row]) + plsc.load_gather(ib_v, [row])
            flat0 = row * NF
            for f in range(NF):
                acc = acc + (plsc.load_gather(ur_v, [flat0 + f])
                             * plsc.load_gather(ir_v, [flat0 + f]))
            out_v[pl.ds(g * _L, _L)] = acc
            return carry

        lax.fori_loop(0, n_groups, dot_body, 0)

        pltpu.sync_copy(out_v, out_hbm.at[pl.ds(base, b_per_w)])

    return k2


def kernel(user_idx, item_idx, user_embedding, item_embedding, user_bias, item_bias):
    n_users, n_items = user_embedding.shape[0], item_embedding.shape[0]
    uT3 = user_embedding.T.reshape(NF // 8, 8, n_users)
    iT3 = item_embedding.T.reshape(NF // 8, 8, n_items)
    ru, ri = _sweep_kernel(n_users, n_items)(user_idx, item_idx, uT3, iT3)
    return _combine_kernel()(user_idx, item_idx, ru, ri,
                             user_bias.reshape(-1), item_bias.reshape(-1))


# speedup vs baseline: 2.1000x; 1.0985x over previous
import functools

import jax
import jax.numpy as jnp
from jax import lax
from jax.experimental import pallas as pl
from jax.experimental.pallas import tpu as pltpu
from jax.experimental.pallas import tpu_sc as plsc

BATCH = 16384
NF = 32
CHUNK = 128
WB = 16
SEL_CAP = 1040
IDX_CHUNK = 2048
RING = 16

_info = plsc.get_sparse_core_info()
_NC, _NS, _L = _info.num_cores, _info.num_subcores, _info.num_lanes
_NW = _NC * _NS


def _sweep_kernel(n_users, n_items):
    nt_u = -(-n_users // CHUNK)
    nt_i = -(-n_items // CHUNK)
    wpw_u = -(-nt_u // _NW)
    wpw_i = -(-nt_i // _NW)
    nb_u = -(-wpw_u // WB)
    nb_i = -(-wpw_i // WB)
    mesh = plsc.VectorSubcoreMesh(core_axis_name="c", subcore_axis_name="s")

    @functools.partial(
        pl.kernel,
        mesh=mesh,
        out_type=(jax.ShapeDtypeStruct((BATCH * NF,), jnp.float32),
                  jax.ShapeDtypeStruct((BATCH * NF,), jnp.float32)),
        compiler_params=pltpu.CompilerParams(
            needs_layout_passes=False, use_tc_tiling_on_sc=True),
        scratch_types=[
            pltpu.VMEM((IDX_CHUNK,), jnp.int32),
            pltpu.VMEM((IDX_CHUNK,), jnp.int32),
            pltpu.VMEM((SEL_CAP,), jnp.int32),
            pltpu.VMEM((SEL_CAP,), jnp.int32),
            pltpu.VMEM((SEL_CAP,), jnp.int32),
            pltpu.VMEM((SEL_CAP,), jnp.int32),
            pltpu.VMEM((NF, WB * CHUNK), jnp.float32),
            pltpu.VMEM((RING * NF,), jnp.float32),
            pltpu.SMEM((8,), jnp.int32),
            pltpu.SemaphoreType.DMA,
            pltpu.SemaphoreType.DMA,
        ],
    )
    def k1(uidx_hbm, iidx_hbm, uT3_hbm, iT3_hbm, ru_hbm, ri_hbm,
           uix_v, iix_v, selnu_v, selbu_v, selni_v, selbi_v,
           stage_v, rb_v, cnt_s, sem, sem2):
        wid = lax.axis_index("s") * _NC + lax.axis_index("c")
        lane = lax.iota(jnp.int32, _L)

        nt_u_c, nt_i_c = nt_u, nt_i
        lo_u = wid * wpw_u
        hi_u = jnp.minimum(nt_u_c, lo_u + wpw_u)
        lo_i = wid * wpw_i
        hi_i = jnp.minimum(nt_i_c, lo_i + wpw_i)

        def sel_chunk(ci, cnts):
            pltpu.sync_copy(uidx_hbm.at[pl.ds(ci * IDX_CHUNK, IDX_CHUNK)],
                            uix_v)
            pltpu.sync_copy(iidx_hbm.at[pl.ds(ci * IDX_CHUNK, IDX_CHUNK)],
                            iix_v)

            def sel_vec(j, cnts):
                cu, cit = cnts
                nu = uix_v[pl.ds(j * _L, _L)]
                ni = iix_v[pl.ds(j * _L, _L)]
                wu = nu >> 7
                wi = ni >> 7
                mu = (wu >= lo_u) & (wu < hi_u)
                mi_ = (wi >= lo_i) & (wi < hi_i)
                bv = ci * IDX_CHUNK + j * _L + lane
                plsc.store_compressed(selnu_v.at[pl.ds(cu, _L)], nu, mask=mu)
                plsc.store_compressed(selbu_v.at[pl.ds(cu, _L)], bv, mask=mu)
                plsc.store_compressed(selni_v.at[pl.ds(cit, _L)], ni, mask=mi_)
                plsc.store_compressed(selbi_v.at[pl.ds(cit, _L)], bv, mask=mi_)
                return (cu + plsc.all_reduce_population_count(mu)[0],
                        cit + plsc.all_reduce_population_count(mi_)[0])

            return lax.fori_loop(0, IDX_CHUNK // _L, sel_vec, cnts)

        cnt_u, cnt_i = lax.fori_loop(0, BATCH // IDX_CHUNK, sel_chunk, (0, 0))

        def sweep(tbl3, nt, lo, hi, nb, rout, seln_v, selb_v, cnt):
            def batch_body(bi, carry):
                win0 = lo + bi * WB
                win0c = jnp.minimum(win0, nt - WB)
                col0 = pl.multiple_of(win0c * CHUNK, CHUNK)
                cps = [pltpu.async_copy(
                    tbl3.at[ft, :, pl.ds(col0, WB * CHUNK)],
                    stage_v.at[pl.ds(ft * 8, 8), :], sem2)
                    for ft in range(NF // 8)]
                for cp in cps:
                    cp.wait()
                wend = jnp.minimum(win0 + WB, hi)

                def scan_vec(v, carry):
                    nv = seln_v[pl.ds(v * _L, _L)]
                    bv = selb_v[pl.ds(v * _L, _L)]
                    wv = nv >> 7
                    m = ((v * _L + lane) < cnt) & (wv >= win0) & (wv < wend)
                    nhits = plsc.all_reduce_population_count(m)[0]
                    mi = m.astype(jnp.int32)

                    @pl.when(nhits > 0)
                    def _():
                        for k in range(_L):
                            @pl.when(mi[k] != 0)
                            def _():
                                n_s = nv[k]
                                b_s = bv[k]
                                col = jnp.full((_L,), n_s - win0c * CHUNK,
                                               jnp.int32)
                                r0 = plsc.load_gather(stage_v, [lane, col])
                                r1 = plsc.load_gather(stage_v,
                                                      [lane + _L, col])
                                h = cnt_s[0]

                                @pl.when(h >= RING)
                                def _():
                                    pltpu.make_async_copy(
                                        uidx_hbm.at[pl.ds(0, NF)],
                                        rb_v.at[pl.ds(0, NF)], sem).wait()

                                off = (h & (RING - 1)) * NF
                                rb_v[pl.ds(off, _L)] = r0
                                rb_v[pl.ds(off + _L, _L)] = r1
                                pltpu.async_copy(
                                    rb_v.at[pl.ds(off, NF)],
                                    rout.at[pl.ds(b_s * NF, NF)], sem)
                                cnt_s[0] = h + 1
                    return carry

                n_scan = (cnt + _L - 1) >> 4
                return lax.fori_loop(0, n_scan, scan_vec, carry)

            lax.fori_loop(0, nb, batch_body, 0)

        cnt_s[0] = 0
        sweep(iT3_hbm, nt_i, lo_i, hi_i, nb_i, ri_hbm, selni_v, selbi_v,
              cnt_i)
        sweep(uT3_hbm, nt_u, lo_u, hi_u, nb_u, ru_hbm, selnu_v, selbu_v,
              cnt_u)

        total = cnt_s[0]

        def drain_body(i, carry):
            @pl.when(i < jnp.minimum(total, RING))
            def _():
                pltpu.make_async_copy(uidx_hbm.at[pl.ds(0, NF)],
                                      rb_v.at[pl.ds(0, NF)], sem).wait()
            return carry

        lax.fori_loop(0, RING, drain_body, 0)

    return k1


def _combine_kernel():
    b_per_w = BATCH // _NW
    n_chunks = b_per_w // CHUNK
    n_groups = b_per_w // _L
    mesh = plsc.VectorSubcoreMesh(core_axis_name="c", subcore_axis_name="s")

    @functools.partial(
        pl.kernel,
        mesh=mesh,
        out_type=jax.ShapeDtypeStruct((BATCH,), jnp.float32),
        compiler_params=pltpu.CompilerParams(needs_layout_passes=False),
        scratch_types=[
            pltpu.VMEM((n_chunks, CHUNK), jnp.int32),
            pltpu.VMEM((n_chunks, CHUNK), jnp.int32),
            pltpu.VMEM((b_per_w * NF,), jnp.float32),
            pltpu.VMEM((b_per_w * NF,), jnp.float32),
            pltpu.VMEM((b_per_w,), jnp.float32),
            pltpu.VMEM((b_per_w,), jnp.float32),
            pltpu.VMEM((b_per_w,), jnp.float32),
            pltpu.SemaphoreType.DMA,
        ],
    )
    def k2(uidx_hbm, iidx_hbm, ru_hbm, ri_hbm, ub_hbm, ib_hbm, out_hbm,
           uidx_v, iidx_v, ur_v, ir_v, ub_v, ib_v, out_v, sem):
        wid = lax.axis_index("s") * _NC + lax.axis_index("c")
        base = wid * b_per_w

        for c in range(n_chunks):
            pltpu.sync_copy(uidx_hbm.at[pl.ds(base + c * CHUNK, CHUNK)],
                            uidx_v.at[c])
            pltpu.sync_copy(iidx_hbm.at[pl.ds(base + c * CHUNK, CHUNK)],
                            iidx_v.at[c])

        copies = [
            pltpu.async_copy(ru_hbm.at[pl.ds(base * NF, b_per_w * NF)],
                             ur_v, sem),
            pltpu.async_copy(ri_hbm.at[pl.ds(base * NF, b_per_w * NF)],
                             ir_v, sem),
        ]
        for c in range(n_chunks):
            dst = pl.ds(c * CHUNK, CHUNK)
            copies.append(pltpu.async_copy(
                ub_hbm.at[uidx_v.at[c]], ub_v.at[dst], sem))
            copies.append(pltpu.async_copy(
                ib_hbm.at[iidx_v.at[c]], ib_v.at[dst], sem))
        for cp in copies:
            cp.wait()

        lane = lax.iota(jnp.int32, _L)

        def dot_body(g, carry):
            row = g * _L + lane
            acc = plsc.load_gather(ub_v, [row]) + plsc.load_gather(ib_v, [row])
            flat0 = row * NF
            for f in range(NF):
                acc = acc + (plsc.load_gather(ur_v, [flat0 + f])
                             * plsc.load_gather(ir_v, [flat0 + f]))
            out_v[pl.ds(g * _L, _L)] = acc
            return carry

        lax.fori_loop(0, n_groups, dot_body, 0)

        pltpu.sync_copy(out_v, out_hbm.at[pl.ds(base, b_per_w)])

    return k2


def kernel(user_idx, item_idx, user_embedding, item_embedding, user_bias, item_bias):
    n_users, n_items = user_embedding.shape[0], item_embedding.shape[0]
    uT3 = user_embedding.T.reshape(NF // 8, 8, n_users)
    iT3 = item_embedding.T.reshape(NF // 8, 8, n_items)
    ru, ri = _sweep_kernel(n_users, n_items)(user_idx, item_idx, uT3, iT3)
    return _combine_kernel()(user_idx, item_idx, ru, ri,
                             user_bias.reshape(-1), item_bias.reshape(-1))
